# baseline (speedup 1.0000x reference)
import jax
import jax.numpy as jnp
from jax.experimental import pallas as pl
from jax.experimental.pallas import tpu as pltpu

_S, _N, _F, _H, _R = 8, 1024, 128, 256, 4


def _leaky(v):
    return jnp.where(v >= 0, v, 0.01 * v)


def _dot(a, b):
    return jnp.dot(a, b, preferred_element_type=jnp.float32)


def _rgcn_last_step_kernel(
    x_ref, adj_ref,
    fc1_w_ref, fc1_b_ref, fc2_w_ref, fc2_b_ref,
    fc3_w_ref, fc3_b_ref, fc4_w_ref, fc4_b_ref,
    g0_ws_ref, g0_wr_ref, g0_b_ref,
    g1_ws_ref, g1_wr_ref, g1_b_ref,
    out_ref,
    h_ref, msg_ref, store_ref, inv_deg_ref, sems,
):
    i = pl.program_id(0)

    @pl.when(i < _R - 1)
    def _stash():
        pltpu.make_async_copy(adj_ref.at[0, 0, 0], store_ref.at[i],
                              sems.at[i]).start()

    @pl.when(i == 0)
    def _fc_head():
        x = x_ref[0, 0]
        h = _leaky(_dot(x, fc1_w_ref[...]) + fc1_b_ref[...])
        h = _leaky(_dot(h, fc2_w_ref[...]) + fc2_b_ref[...])
        h_ref[...] = h
        msg_ref[...] = _dot(h, g0_ws_ref[...]) + g0_b_ref[...]

    adj = adj_ref[0, 0, 0]
    inv_deg = 1.0 / (jnp.sum(adj, axis=1, keepdims=True) + 1e-6)
    inv_deg_ref[i] = inv_deg
    adjn = adj * inv_deg
    h = h_ref[...]
    agg = _dot(adjn, h)
    msg_ref[...] = msg_ref[...] + _dot(agg, g0_wr_ref[i])

    @pl.when(i < _R - 1)
    def _stash_done():
        pltpu.make_async_copy(adj_ref.at[0, 0, 0], store_ref.at[i],
                              sems.at[i]).wait()

    @pl.when(i == _R - 1)
    def _layer1_and_tail():
        h1 = _leaky(msg_ref[...])
        acc = _dot(h1, g1_ws_ref[...]) + g1_b_ref[...]
        for r in range(_R - 1):
            adjn_r = store_ref[r] * inv_deg_ref[r]
            acc = acc + _dot(_dot(adjn_r, h1), g1_wr_ref[r])
        acc = acc + _dot(_dot(adjn, h1), g1_wr_ref[_R - 1])
        h2 = _leaky(acc)
        o = _leaky(_dot(h2, fc3_w_ref[...]) + fc3_b_ref[...])
        out_ref[0] = _dot(o, fc4_w_ref[...]) + fc4_b_ref[...]


def kernel(x, adjs, edgenum, fc1_w, fc1_b, fc2_w, fc2_b, fc3_w, fc3_b,
           fc4_w, fc4_b, g0_wself, g0_wrel, g0_b, g1_wself, g1_wrel, g1_b):
    del edgenum
    last = _S - 1

    def full(shape):
        return pl.BlockSpec(shape, lambda i: tuple(0 for _ in shape))

    in_specs = [
        pl.BlockSpec((1, 1, _N, _F), lambda i: (0, last, 0, 0)),
        pl.BlockSpec((1, 1, 1, _N, _N), lambda i: (0, last, i, 0, 0)),
        full((_F, _H)), full((1, _H)),
        full((_H, _H)), full((1, _H)),
        full((_H, _H)), full((1, _H)),
        full((_H, 1)), full((1, 1)),
        full((_H, _H)), full((_R, _H, _H)), full((1, _H)),
        full((_H, _H)), full((_R, _H, _H)), full((1, _H)),
    ]

    out = pl.pallas_call(
        _rgcn_last_step_kernel,
        out_shape=jax.ShapeDtypeStruct((1, _N, 1), jnp.float32),
        grid=(_R,),
        in_specs=in_specs,
        out_specs=pl.BlockSpec((1, _N, 1), lambda i: (0, 0, 0)),
        scratch_shapes=[
            pltpu.VMEM((_N, _H), jnp.float32),
            pltpu.VMEM((_N, _H), jnp.float32),
            pltpu.VMEM((_R - 1, _N, _N), jnp.float32),
            pltpu.VMEM((_R, _N, 1), jnp.float32),
            pltpu.SemaphoreType.DMA((_R - 1,)),
        ],
        compiler_params=pltpu.CompilerParams(
            vmem_limit_bytes=100 * 1024 * 1024,
        ),
    )(
        x, adjs,
        fc1_w, fc1_b.reshape(1, _H), fc2_w, fc2_b.reshape(1, _H),
        fc3_w, fc3_b.reshape(1, _H),
        fc4_w, fc4_b.reshape(1, 1),
        g0_wself, g0_wrel, g0_b.reshape(1, _H),
        g1_wself, g1_wrel, g1_b.reshape(1, _H),
    )
    return out


# speedup vs baseline: 1.0446x; 1.0446x over previous
import jax
import jax.numpy as jnp
from jax.experimental import pallas as pl
from jax.experimental.pallas import tpu as pltpu

_S, _N, _F, _H, _R = 8, 1024, 128, 256, 4


def _leaky(v):
    return jnp.where(v >= 0, v, 0.01 * v)


def _dot(a, b):
    return jnp.dot(a, b, preferred_element_type=jnp.float32)


def _rgcn_last_step_kernel(
    x_ref, adjs_hbm_ref,
    fc1_w_ref, fc1_b_ref, fc2_w_ref, fc2_b_ref,
    fc3_w_ref, fc3_b_ref, fc4_w_ref, fc4_b_ref,
    g0_ws_ref, g0_wr_ref, g0_b_ref,
    g1_ws_ref, g1_wr_ref, g1_b_ref,
    out_ref,
    adj_ref, sems,
):
    copies = [
        pltpu.make_async_copy(adjs_hbm_ref.at[0, _S - 1, r],
                              adj_ref.at[r], sems.at[r])
        for r in range(_R)
    ]
    for c in copies:
        c.start()

    x = x_ref[0, 0]
    h = _leaky(_dot(x, fc1_w_ref[...]) + fc1_b_ref[...])
    h = _leaky(_dot(h, fc2_w_ref[...]) + fc2_b_ref[...])

    acc = _dot(h, g0_ws_ref[...]) + g0_b_ref[...]
    for r in range(_R):
        copies[r].wait()
        adj = adj_ref[r]
        adjn = adj / (jnp.sum(adj, axis=1, keepdims=True) + 1e-6)
        adj_ref[r] = adjn
        acc = acc + _dot(_dot(adjn, h), g0_wr_ref[r])
    h = _leaky(acc)

    acc = _dot(h, g1_ws_ref[...]) + g1_b_ref[...]
    for r in range(_R):
        acc = acc + _dot(_dot(adj_ref[r], h), g1_wr_ref[r])
    h = _leaky(acc)

    o = _leaky(_dot(h, fc3_w_ref[...]) + fc3_b_ref[...])
    out_ref[0] = _dot(o, fc4_w_ref[...]) + fc4_b_ref[...]


def kernel(x, adjs, edgenum, fc1_w, fc1_b, fc2_w, fc2_b, fc3_w, fc3_b,
           fc4_w, fc4_b, g0_wself, g0_wrel, g0_b, g1_wself, g1_wrel, g1_b):
    del edgenum
    last = _S - 1

    def full(shape):
        return pl.BlockSpec(shape, lambda i: tuple(0 for _ in shape))

    in_specs = [
        pl.BlockSpec((1, 1, _N, _F), lambda i: (0, last, 0, 0)),
        pl.BlockSpec(memory_space=pltpu.MemorySpace.HBM),
        full((_F, _H)), full((1, _H)),
        full((_H, _H)), full((1, _H)),
        full((_H, _H)), full((1, _H)),
        full((_H, 1)), full((1, 1)),
        full((_H, _H)), full((_R, _H, _H)), full((1, _H)),
        full((_H, _H)), full((_R, _H, _H)), full((1, _H)),
    ]

    out = pl.pallas_call(
        _rgcn_last_step_kernel,
        out_shape=jax.ShapeDtypeStruct((1, _N, 1), jnp.float32),
        grid=(1,),
        in_specs=in_specs,
        out_specs=pl.BlockSpec((1, _N, 1), lambda i: (0, 0, 0)),
        scratch_shapes=[
            pltpu.VMEM((_R, _N, _N), jnp.float32),
            pltpu.SemaphoreType.DMA((_R,)),
        ],
        compiler_params=pltpu.CompilerParams(
            vmem_limit_bytes=100 * 1024 * 1024,
        ),
    )(
        x, adjs,
        fc1_w, fc1_b.reshape(1, _H), fc2_w, fc2_b.reshape(1, _H),
        fc3_w, fc3_b.reshape(1, _H),
        fc4_w, fc4_b.reshape(1, 1),
        g0_wself, g0_wrel, g0_b.reshape(1, _H),
        g1_wself, g1_wrel, g1_b.reshape(1, _H),
    )
    return out
